# Initial kernel scaffold; baseline (speedup 1.0000x reference)
import functools

import jax
import jax.numpy as jnp
from jax import lax
from jax.experimental import pallas as pl
from jax.experimental.pallas import tpu as pltpu
from jax.experimental.pallas import tpu_sc as plsc

VOCAB = 50264
DIM = 10
B, L = 4096, 200
N_IDX = B * L


_ROW_BLK = 8192


def _project_body(tab_ref, wt_ref, b_ref, out_ref):
    x = tab_ref[...]
    wt = wt_ref[...]
    out_ref[...] = (
        jnp.dot(x, wt, preferred_element_type=jnp.float32) + b_ref[...]
    )


def _project_table(table, Wt, b2d):
    grid = (VOCAB + _ROW_BLK - 1) // _ROW_BLK
    return pl.pallas_call(
        _project_body,
        grid=(grid,),
        in_specs=[
            pl.BlockSpec((_ROW_BLK, DIM), lambda i: (i, 0)),
            pl.BlockSpec((DIM, DIM), lambda i: (0, 0)),
            pl.BlockSpec((1, DIM), lambda i: (0, 0)),
        ],
        out_specs=pl.BlockSpec((_ROW_BLK, DIM), lambda i: (i, 0)),
        out_shape=jax.ShapeDtypeStruct((VOCAB, DIM), jnp.float32),
    )(table, Wt, b2d)


_NW = 32
_PER_W = N_IDX // _NW
_CH = 3200
_NCHUNK = _PER_W // _CH


@functools.partial(
    pl.kernel,
    mesh=plsc.VectorSubcoreMesh(core_axis_name="c", subcore_axis_name="s"),
    out_type=jax.ShapeDtypeStruct((N_IDX, DIM), jnp.float32),
    scratch_types=[
        pltpu.VMEM((_CH,), jnp.int32),
        pltpu.VMEM((_CH, DIM), jnp.float32),
        pltpu.SemaphoreType.DMA,
    ],
)
def _gather_rows(idx_hbm, proj_hbm, out_hbm, idx_v, rows_v, sem):
    wid = lax.axis_index("s") * 2 + lax.axis_index("c")
    base = wid * _PER_W

    def body(j, carry):
        off = base + j * _CH
        pltpu.sync_copy(idx_hbm.at[pl.ds(off, _CH)], idx_v)
        pltpu.async_copy(proj_hbm.at[idx_v], rows_v, sem).wait()
        pltpu.sync_copy(rows_v, out_hbm.at[pl.ds(off, _CH)])
        return carry

    lax.fori_loop(0, _NCHUNK, body, 0)


def kernel(input_ids, table, W, b):
    Wt = W.T
    b2d = b.reshape(1, DIM)
    proj = _project_table(table, Wt, b2d)
    flat_idx = input_ids.reshape(-1).astype(jnp.int32)
    out_flat = _gather_rows(flat_idx, proj)
    return out_flat.reshape(B, L, DIM)


# speedup vs baseline: 3.1688x; 3.1688x over previous
import functools

import jax
import jax.numpy as jnp
from jax import lax
from jax.experimental import pallas as pl
from jax.experimental.pallas import tpu as pltpu
from jax.experimental.pallas import tpu_sc as plsc

VOCAB = 50264
DIM = 10
DIMP = 16
B, L = 4096, 200
N_IDX = B * L


_ROW_BLK = 8192


def _project_body(tab_ref, wt_ref, b_ref, out_ref):
    x = tab_ref[...]
    wt = wt_ref[...]
    out_ref[...] = (
        jnp.dot(x, wt, preferred_element_type=jnp.float32) + b_ref[...]
    )


def _project_table(table, Wt, b2d):
    grid = (VOCAB + _ROW_BLK - 1) // _ROW_BLK
    return pl.pallas_call(
        _project_body,
        grid=(grid,),
        in_specs=[
            pl.BlockSpec((_ROW_BLK, DIM), lambda i: (i, 0)),
            pl.BlockSpec((DIM, DIMP), lambda i: (0, 0)),
            pl.BlockSpec((1, DIMP), lambda i: (0, 0)),
        ],
        out_specs=pl.BlockSpec((_ROW_BLK, DIMP), lambda i: (i, 0)),
        out_shape=jax.ShapeDtypeStruct((VOCAB, DIMP), jnp.float32),
    )(table, Wt, b2d)


_NW = 32
_PER_W = N_IDX // _NW
_SUB = 128
_ROWS_PER_W = _PER_W // _SUB
_CH = 3200
_PER_CH = _CH // _SUB
_NCHUNK = _PER_W // _CH


@functools.partial(
    pl.kernel,
    mesh=plsc.VectorSubcoreMesh(core_axis_name="c", subcore_axis_name="s"),
    out_type=jax.ShapeDtypeStruct((N_IDX, DIMP), jnp.float32),
    compiler_params=pltpu.CompilerParams(use_tc_tiling_on_sc=False),
    scratch_types=[
        pltpu.VMEM((_ROWS_PER_W, _SUB), jnp.int32),
        pltpu.VMEM((_CH, DIMP), jnp.float32),
        pltpu.SemaphoreType.DMA,
    ],
)
def _gather_rows(idx_hbm, proj_hbm, out_hbm, idx_v, rows_v, sem):
    wid = lax.axis_index("s") * 2 + lax.axis_index("c")
    pltpu.sync_copy(idx_hbm.at[pl.ds(wid * _ROWS_PER_W, _ROWS_PER_W)], idx_v)

    def chunk(j, carry):
        def sub(i, c):
            pltpu.async_copy(
                proj_hbm.at[idx_v.at[j * _PER_CH + i]],
                rows_v.at[pl.ds(i * _SUB, _SUB)],
                sem,
            ).wait()
            return c

        lax.fori_loop(0, _PER_CH, sub, 0)
        pltpu.sync_copy(rows_v, out_hbm.at[pl.ds(wid * _PER_W + j * _CH, _CH)])
        return carry

    lax.fori_loop(0, _NCHUNK, chunk, 0)


_UNPAD_BLK = 16384


def _unpad_body(in_ref, out_ref):
    out_ref[...] = in_ref[:, :DIM]


def _unpad(out_pad):
    return pl.pallas_call(
        _unpad_body,
        grid=(N_IDX // _UNPAD_BLK,),
        in_specs=[pl.BlockSpec((_UNPAD_BLK, DIMP), lambda i: (i, 0))],
        out_specs=pl.BlockSpec((_UNPAD_BLK, DIM), lambda i: (i, 0)),
        out_shape=jax.ShapeDtypeStruct((N_IDX, DIM), jnp.float32),
    )(out_pad)


def kernel(input_ids, table, W, b):
    Wt = jnp.zeros((DIM, DIMP), jnp.float32).at[:, :DIM].set(W.T)
    b2d = jnp.zeros((1, DIMP), jnp.float32).at[:, :DIM].set(b)
    proj = _project_table(table, Wt, b2d)
    idx2d = input_ids.reshape(N_IDX // _SUB, _SUB).astype(jnp.int32)
    out_pad = _gather_rows(idx2d, proj)
    return _unpad(out_pad).reshape(B, L, DIM)


# speedup vs baseline: 3.6214x; 1.1428x over previous
import functools

import jax
import jax.numpy as jnp
from jax import lax
from jax.experimental import pallas as pl
from jax.experimental.pallas import tpu as pltpu
from jax.experimental.pallas import tpu_sc as plsc

VOCAB = 50264
DIM = 10
DIMP = 16
B, L = 4096, 200
N_IDX = B * L


_ROW_BLK = 8192


def _project_body(tab_ref, wt_ref, b_ref, out_ref):
    x = tab_ref[...]
    wt = wt_ref[...]
    out_ref[...] = (
        jnp.dot(x, wt, preferred_element_type=jnp.float32) + b_ref[...]
    )


def _project_table(table, Wt, b2d):
    grid = (VOCAB + _ROW_BLK - 1) // _ROW_BLK
    return pl.pallas_call(
        _project_body,
        grid=(grid,),
        in_specs=[
            pl.BlockSpec((_ROW_BLK, DIM), lambda i: (i, 0)),
            pl.BlockSpec((DIM, DIMP), lambda i: (0, 0)),
            pl.BlockSpec((1, DIMP), lambda i: (0, 0)),
        ],
        out_specs=pl.BlockSpec((_ROW_BLK, DIMP), lambda i: (i, 0)),
        out_shape=jax.ShapeDtypeStruct((VOCAB, DIMP), jnp.float32),
    )(table, Wt, b2d)


_NW = 32
_PER_W = N_IDX // _NW
_SUB = 128
_ROWS_PER_W = _PER_W // _SUB
_CH = 3200
_PER_CH = _CH // _SUB
_NCHUNK = _PER_W // _CH


@functools.partial(
    pl.kernel,
    mesh=plsc.VectorSubcoreMesh(core_axis_name="c", subcore_axis_name="s"),
    out_type=jax.ShapeDtypeStruct((N_IDX, DIMP), jnp.float32),
    compiler_params=pltpu.CompilerParams(use_tc_tiling_on_sc=False),
    scratch_types=[
        pltpu.VMEM((_ROWS_PER_W, _SUB), jnp.int32),
        pltpu.VMEM((_CH, DIMP), jnp.float32),
        pltpu.SemaphoreType.DMA,
    ],
)
def _gather_rows(idx_hbm, proj_hbm, out_hbm, idx_v, rows_v, sem):
    wid = lax.axis_index("s") * 2 + lax.axis_index("c")
    pltpu.sync_copy(idx_hbm.at[pl.ds(wid * _ROWS_PER_W, _ROWS_PER_W)], idx_v)

    def chunk(j, carry):
        def fire(i, c):
            pltpu.async_copy(
                proj_hbm.at[idx_v.at[j * _PER_CH + i]],
                rows_v.at[pl.ds(i * _SUB, _SUB)],
                sem,
            )
            return c

        lax.fori_loop(0, _PER_CH, fire, 0)

        def drain(i, c):
            pltpu.make_async_copy(
                proj_hbm.at[idx_v.at[j * _PER_CH + i]],
                rows_v.at[pl.ds(i * _SUB, _SUB)],
                sem,
            ).wait()
            return c

        lax.fori_loop(0, _PER_CH, drain, 0)
        pltpu.sync_copy(rows_v, out_hbm.at[pl.ds(wid * _PER_W + j * _CH, _CH)])
        return carry

    lax.fori_loop(0, _NCHUNK, chunk, 0)


_UNPAD_BLK = 16384


def _unpad_body(in_ref, out_ref):
    out_ref[...] = in_ref[:, :DIM]


def _unpad(out_pad):
    return pl.pallas_call(
        _unpad_body,
        grid=(N_IDX // _UNPAD_BLK,),
        in_specs=[pl.BlockSpec((_UNPAD_BLK, DIMP), lambda i: (i, 0))],
        out_specs=pl.BlockSpec((_UNPAD_BLK, DIM), lambda i: (i, 0)),
        out_shape=jax.ShapeDtypeStruct((N_IDX, DIM), jnp.float32),
    )(out_pad)


def kernel(input_ids, table, W, b):
    Wt = jnp.zeros((DIM, DIMP), jnp.float32).at[:, :DIM].set(W.T)
    b2d = jnp.zeros((1, DIMP), jnp.float32).at[:, :DIM].set(b)
    proj = _project_table(table, Wt, b2d)
    idx2d = input_ids.reshape(N_IDX // _SUB, _SUB).astype(jnp.int32)
    out_pad = _gather_rows(idx2d, proj)
    return _unpad(out_pad).reshape(B, L, DIM)


# speedup vs baseline: 5.2956x; 1.4623x over previous
import functools

import jax
import jax.numpy as jnp
from jax import lax
from jax.experimental import pallas as pl
from jax.experimental.pallas import tpu as pltpu
from jax.experimental.pallas import tpu_sc as plsc

VOCAB = 50264
DIM = 10
DIMP = 16
B, L = 4096, 200
N_IDX = B * L


_ROW_BLK = 8192


def _project_body(tab_ref, wt_ref, b_ref, out_ref):
    x = tab_ref[...]
    wt = wt_ref[...]
    out_ref[...] = (
        jnp.dot(x, wt, preferred_element_type=jnp.float32) + b_ref[...]
    )


def _project_table(table, Wt, b2d):
    grid = (VOCAB + _ROW_BLK - 1) // _ROW_BLK
    return pl.pallas_call(
        _project_body,
        grid=(grid,),
        in_specs=[
            pl.BlockSpec((_ROW_BLK, DIM), lambda i: (i, 0)),
            pl.BlockSpec((DIM, DIMP), lambda i: (0, 0)),
            pl.BlockSpec((1, DIMP), lambda i: (0, 0)),
        ],
        out_specs=pl.BlockSpec((_ROW_BLK, DIMP), lambda i: (i, 0)),
        out_shape=jax.ShapeDtypeStruct((VOCAB, DIMP), jnp.float32),
    )(table, Wt, b2d)


_NW = 32
_PER_W = N_IDX // _NW
_SUB = 128
_ROWS_PER_W = _PER_W // _SUB
_CH = 3200
_PER_CH = _CH // _SUB
_NCHUNK = _PER_W // _CH


@functools.partial(
    pl.kernel,
    mesh=plsc.VectorSubcoreMesh(core_axis_name="c", subcore_axis_name="s"),
    out_type=jax.ShapeDtypeStruct((N_IDX, DIMP), jnp.float32),
    compiler_params=pltpu.CompilerParams(use_tc_tiling_on_sc=False),
    scratch_types=[
        pltpu.VMEM((_ROWS_PER_W, _SUB), jnp.int32),
        pltpu.VMEM((_CH, DIMP), jnp.float32),
        pltpu.SemaphoreType.DMA,
    ],
)
def _gather_rows(idx_hbm, proj_hbm, out_hbm, idx_v, rows_v, sem):
    wid = lax.axis_index("s") * 2 + lax.axis_index("c")
    pltpu.sync_copy(idx_hbm.at[pl.ds(wid * _ROWS_PER_W, _ROWS_PER_W)], idx_v)

    def chunk(j, carry):
        def fire(i, c):
            pltpu.async_copy(
                proj_hbm.at[idx_v.at[j * _PER_CH + i]],
                rows_v.at[pl.ds(i * _SUB, _SUB)],
                sem,
            )
            return c

        lax.fori_loop(0, _PER_CH, fire, 0)

        def drain(i, c):
            pltpu.make_async_copy(
                proj_hbm.at[idx_v.at[j * _PER_CH + i]],
                rows_v.at[pl.ds(i * _SUB, _SUB)],
                sem,
            ).wait()
            return c

        lax.fori_loop(0, _PER_CH, drain, 0)
        pltpu.sync_copy(rows_v, out_hbm.at[pl.ds(wid * _PER_W + j * _CH, _CH)])
        return carry

    lax.fori_loop(0, _NCHUNK, chunk, 0)


_UNPAD_BLK = 16384


def _unpad_body(in_ref, out_ref):
    out_ref[...] = in_ref[:, :DIM]


def _unpad(out_pad):
    return pl.pallas_call(
        _unpad_body,
        grid=(N_IDX // _UNPAD_BLK,),
        in_specs=[pl.BlockSpec((_UNPAD_BLK, DIMP), lambda i: (i, 0))],
        out_specs=pl.BlockSpec((_UNPAD_BLK, DIM), lambda i: (i, 0)),
        out_shape=jax.ShapeDtypeStruct((N_IDX, DIM), jnp.float32),
    )(out_pad)


def kernel(input_ids, table, W, b):
    Wt = jnp.zeros((DIM, DIMP), jnp.float32).at[:, :DIM].set(W.T)
    b2d = jnp.zeros((1, DIMP), jnp.float32).at[:, :DIM].set(b)
    proj = _project_table(table, Wt, b2d)
    idx2d = input_ids.reshape(N_IDX // _SUB, _SUB).astype(jnp.int32)
    out_pad = _gather_rows(idx2d, proj)
    return out_pad[:, :DIM].reshape(B, L, DIM)
